# baseline (speedup 1.0000x reference)
import functools

import jax
import jax.numpy as jnp
from jax.experimental import pallas as pl
from jax.experimental.pallas import tpu as pltpu

E = 8
TOP_K = 2
LANES = 128


def _router_body(x_ref, gate_ref, logits_ref, wpe_ref):
    x = x_ref[...]
    gate = gate_ref[...]
    logits = jax.lax.dot_general(
        x, gate, (((1,), (1,)), ((), ())), preferred_element_type=jnp.float32
    )
    T = logits.shape[0]
    lane = jax.lax.broadcasted_iota(jnp.int32, (T, LANES), 1)
    neg = jnp.float32(-1e30)
    masked = jnp.where(lane < E, logits, neg)
    m = jnp.max(masked, axis=1, keepdims=True)
    ex = jnp.exp(masked - m)
    s = jnp.sum(ex, axis=1, keepdims=True)
    p = ex / s
    m1 = jnp.max(p, axis=1, keepdims=True)
    i1 = jnp.min(jnp.where(p == m1, lane, LANES), axis=1, keepdims=True)
    p_rest = jnp.where(lane == i1, jnp.float32(-1.0), p)
    m2 = jnp.max(p_rest, axis=1, keepdims=True)
    i2 = jnp.min(jnp.where(p_rest == m2, lane, LANES), axis=1, keepdims=True)
    denom = m1 + m2
    w1n = m1 / denom
    w2n = m2 / denom
    wpe = jnp.where(lane == i1, w1n, 0.0) + jnp.where(lane == i2, w2n, 0.0)
    logits_ref[...] = logits
    wpe_ref[...] = wpe


def _router(x, gate_pad):
    T = x.shape[0]
    return pl.pallas_call(
        _router_body,
        out_shape=(
            jax.ShapeDtypeStruct((T, LANES), jnp.float32),
            jax.ShapeDtypeStruct((T, LANES), jnp.float32),
        ),
    )(x, gate_pad)


def _ffn_body(x_ref, w1_ref, w3_ref, w2_ref, wpe_ref, out_ref):
    e = pl.program_id(0)
    f = pl.program_id(1)
    t = pl.program_id(2)
    TM = x_ref.shape[0]
    base = t * TM

    @pl.when((e == 0) & (f == 0))
    def _():
        out_ref[pl.ds(base, TM), :] = jnp.zeros_like(x_ref)

    x = x_ref[...]
    w1 = w1_ref[0]
    w3 = w3_ref[0]
    w2 = w2_ref[0]
    h1 = jax.lax.dot_general(x, w1, (((1,), (1,)), ((), ())),
                             preferred_element_type=jnp.float32)
    h3 = jax.lax.dot_general(x, w3, (((1,), (1,)), ((), ())),
                             preferred_element_type=jnp.float32)
    h = (h1 * jax.lax.logistic(h1)) * h3
    y = jax.lax.dot_general(h, w2, (((1,), (1,)), ((), ())),
                            preferred_element_type=jnp.float32)
    w = wpe_ref[0, 0, pl.ds(base, TM)]
    out_ref[pl.ds(base, TM), :] += y * w[:, None]


def _ffn(x, w1, w3, w2, wpe_r):
    T, H = x.shape
    FFN = w1.shape[1]
    TM = 256
    FC = 512
    grid = (E, FFN // FC, T // TM)
    return pl.pallas_call(
        _ffn_body,
        grid=grid,
        in_specs=[
            pl.BlockSpec((TM, H), lambda e, f, t: (t, 0)),
            pl.BlockSpec((1, FC, H), lambda e, f, t: (e, f, 0)),
            pl.BlockSpec((1, FC, H), lambda e, f, t: (e, f, 0)),
            pl.BlockSpec((1, H, FC), lambda e, f, t: (e, 0, f)),
            pl.BlockSpec((1, 1, T), lambda e, f, t: (e, 0, 0)),
        ],
        out_specs=pl.BlockSpec((T, H), lambda e, f, t: (0, 0)),
        out_shape=jax.ShapeDtypeStruct((T, H), jnp.float32),
        compiler_params=pltpu.CompilerParams(
            dimension_semantics=("arbitrary", "arbitrary", "arbitrary"),
        ),
    )(x, w1, w3, w2, wpe_r)


@jax.jit
def kernel(hidden_states, gate_w, w1, w2, w3):
    B, S, H = hidden_states.shape
    x = hidden_states.reshape(-1, H)
    T = x.shape[0]
    gate_pad = jnp.zeros((LANES, H), jnp.float32).at[:E].set(gate_w)
    logits_pad, wpe = _router(x, gate_pad)
    router_logits = logits_pad[:, :E]
    wpe_r = wpe[:, :E].T.reshape(E, 1, T)
    final = _ffn(x, w1, w3, w2, wpe_r)
    return final.reshape(B, S, H), router_logits


# speedup vs baseline: 1.1721x; 1.1721x over previous
import functools

import jax
import jax.numpy as jnp
from jax import lax
from jax.experimental import pallas as pl
from jax.experimental.pallas import tpu as pltpu
from jax.experimental.pallas import tpu_sc as plsc

E = 8
TOP_K = 2
LANES = 128
T = 2048
H = 1024
FFN = 2048
TILE = 128
NP = T * TOP_K + E * TILE
NT = NP // TILE
NWORK = 32


def _router_body(x_ref, gate_ref, logits_ref, pos1_ref, pos2_ref,
                 wn1_ref, wn2_ref, te_ref, valid_ref):
    x = x_ref[...]
    gate = gate_ref[...]
    logits = lax.dot_general(x, gate, (((1,), (1,)), ((), ())),
                             preferred_element_type=jnp.float32)
    lane = lax.broadcasted_iota(jnp.int32, (T, LANES), 1)
    neg = jnp.float32(-1e30)
    masked = jnp.where(lane < E, logits, neg)
    m = jnp.max(masked, axis=1, keepdims=True)
    ex = jnp.exp(masked - m)
    p = ex / jnp.sum(ex, axis=1, keepdims=True)
    m1 = jnp.max(p, axis=1, keepdims=True)
    i1 = jnp.min(jnp.where(p == m1, lane, LANES), axis=1, keepdims=True)
    p_rest = jnp.where(lane == i1, jnp.float32(-1.0), p)
    m2 = jnp.max(p_rest, axis=1, keepdims=True)
    i2 = jnp.min(jnp.where(p_rest == m2, lane, LANES), axis=1, keepdims=True)
    denom = m1 + m2
    sel1 = lane == i1
    sel2 = lane == i2
    mask = jnp.where(sel1 | sel2, jnp.float32(1.0), 0.0)
    r_io = lax.broadcasted_iota(jnp.int32, (T, T), 0)
    c_io = lax.broadcasted_iota(jnp.int32, (T, T), 1)
    tril = jnp.where(r_io > c_io, jnp.float32(1.0), 0.0)
    cum = lax.dot_general(tril, mask, (((1,), (0,)), ((), ())),
                          preferred_element_type=jnp.float32)
    counts = jnp.sum(mask, axis=0, keepdims=True)
    padded = jnp.ceil(counts / TILE) * TILE
    ri = lax.broadcasted_iota(jnp.int32, (LANES, LANES), 0)
    ci = lax.broadcasted_iota(jnp.int32, (LANES, LANES), 1)
    upper = jnp.where(ri < ci, jnp.float32(1.0), 0.0)
    offs = lax.dot_general(padded, upper, (((1,), (0,)), ((), ())),
                           preferred_element_type=jnp.float32)
    ends = offs + padded
    pos = offs + cum
    pos1 = jnp.sum(jnp.where(sel1, pos, 0.0), axis=1, keepdims=True)
    pos2 = jnp.sum(jnp.where(sel2, pos, 0.0), axis=1, keepdims=True)
    tstart = (lax.broadcasted_iota(jnp.int32, (LANES, LANES), 0)
              * TILE).astype(jnp.float32)
    lane2 = lax.broadcasted_iota(jnp.int32, (LANES, LANES), 1)
    hit = jnp.where((tstart >= jnp.broadcast_to(ends, (LANES, LANES)))
                    & (lane2 < E), jnp.float32(1.0), 0.0)
    cnt = jnp.sum(hit, axis=1, keepdims=True).astype(jnp.int32)
    logits_ref[...] = logits
    pos1_ref[...] = pos1.astype(jnp.int32)
    pos2_ref[...] = pos2.astype(jnp.int32)
    wn1_ref[...] = m1 / denom
    wn2_ref[...] = m2 / denom
    te_ref[...] = jnp.minimum(cnt, E - 1)
    valid_ref[...] = jnp.where(cnt < E, 1, 0).astype(jnp.int32)


def _router(x, gate_pad):
    return pl.pallas_call(
        _router_body,
        out_shape=(
            jax.ShapeDtypeStruct((T, LANES), jnp.float32),
            jax.ShapeDtypeStruct((T, 1), jnp.int32),
            jax.ShapeDtypeStruct((T, 1), jnp.int32),
            jax.ShapeDtypeStruct((T, 1), jnp.float32),
            jax.ShapeDtypeStruct((T, 1), jnp.float32),
            jax.ShapeDtypeStruct((LANES, 1), jnp.int32),
            jax.ShapeDtypeStruct((LANES, 1), jnp.int32),
        ),
    )(x, gate_pad)


def _sc_mesh():
    return plsc.VectorSubcoreMesh(core_axis_name="c", subcore_axis_name="s",
                                  num_cores=2, num_subcores=16)


@functools.lru_cache(maxsize=None)
def _sc_meta_kernel():
    return functools.partial(
        pl.kernel,
        mesh=_sc_mesh(),
        out_type=(
            jax.ShapeDtypeStruct((NP,), jnp.int32),
            jax.ShapeDtypeStruct((NP,), jnp.float32),
        ),
        scratch_types=[
            pltpu.VMEM((T,), jnp.int32),
            pltpu.VMEM((T,), jnp.int32),
            pltpu.VMEM((T,), jnp.float32),
            pltpu.VMEM((T,), jnp.float32),
            pltpu.VMEM((NP,), jnp.int32),
            pltpu.VMEM((NP,), jnp.float32),
        ],
        compiler_params=pltpu.CompilerParams(needs_layout_passes=False),
    )(_sc_meta_body)


def _sc_meta(p1, p2, a1, a2):
    return _sc_meta_kernel()(p1, p2, a1, a2)


def _sc_meta_body(p1_hbm, p2_hbm, a1_hbm, a2_hbm, tok_hbm, wgt_hbm,
                  p1_v, p2_v, a1_v, a2_v, tok_v, wgt_v):
    wid = lax.axis_index("s") * 2 + lax.axis_index("c")

    @pl.when(wid == 0)
    def _():
        pltpu.sync_copy(p1_hbm, p1_v)
        pltpu.sync_copy(p2_hbm, p2_v)
        pltpu.sync_copy(a1_hbm, a1_v)
        pltpu.sync_copy(a2_hbm, a2_v)

        def init(i, carry):
            tok_v[pl.ds(i * 16, 16)] = jnp.zeros((16,), jnp.int32)
            wgt_v[pl.ds(i * 16, 16)] = jnp.zeros((16,), jnp.float32)
            return carry

        lax.fori_loop(0, NP // 16, init, 0)

        def scat(i, carry):
            sl = pl.ds(i * 16, 16)
            tvec = lax.iota(jnp.int32, 16) + i * 16
            plsc.store_scatter(tok_v, [p1_v[sl]], tvec)
            plsc.store_scatter(wgt_v, [p1_v[sl]], a1_v[sl])
            plsc.store_scatter(tok_v, [p2_v[sl]], tvec)
            plsc.store_scatter(wgt_v, [p2_v[sl]], a2_v[sl])
            return carry

        lax.fori_loop(0, T // 16, scat, 0)
        pltpu.sync_copy(tok_v, tok_hbm)
        pltpu.sync_copy(wgt_v, wgt_hbm)


_SLOTS_PER = NP // NWORK
_GCH = 32


@functools.lru_cache(maxsize=None)
def _sc_gather_kernel():
    return functools.partial(
        pl.kernel,
        mesh=_sc_mesh(),
        out_type=jax.ShapeDtypeStruct((NP, H), jnp.float32),
        scratch_types=[
            pltpu.VMEM((_SLOTS_PER,), jnp.int32),
            pltpu.VMEM((_GCH, H), jnp.float32),
            pltpu.SemaphoreType.DMA,
        ],
        compiler_params=pltpu.CompilerParams(needs_layout_passes=False),
    )(_sc_gather_body)


def _sc_gather(x, tok):
    return _sc_gather_kernel()(x, tok)


def _sc_gather_body(x_hbm, tok_hbm, xs_hbm, idx_v, rows_v, sem):
    wid = lax.axis_index("s") * 2 + lax.axis_index("c")
    base = wid * _SLOTS_PER
    pltpu.sync_copy(tok_hbm.at[pl.ds(base, _SLOTS_PER)], idx_v)

    def step(c, carry):
        pltpu.async_copy(
            x_hbm.at[idx_v.at[pl.ds(c * _GCH, _GCH)]], rows_v, sem
        ).wait()
        pltpu.sync_copy(rows_v, xs_hbm.at[pl.ds(base + c * _GCH, _GCH)])
        return carry

    lax.fori_loop(0, _SLOTS_PER // _GCH, step, 0)


def _ffn_body(pf_ref, x_ref, w1_ref, w3_ref, w2_ref, wgt_ref, out_ref):
    i = pl.program_id(0)

    @pl.when(pf_ref[1, i] == 1)
    def _():
        x = x_ref[...]
        w1 = w1_ref[0]
        w3 = w3_ref[0]
        w2 = w2_ref[0]
        h1 = lax.dot_general(x, w1, (((1,), (1,)), ((), ())),
                             preferred_element_type=jnp.float32)
        h3 = lax.dot_general(x, w3, (((1,), (1,)), ((), ())),
                             preferred_element_type=jnp.float32)
        h = (h1 * lax.logistic(h1)) * h3
        y = lax.dot_general(h, w2, (((1,), (1,)), ((), ())),
                            preferred_element_type=jnp.float32)
        out_ref[...] = y * wgt_ref[...]


def _ffn_grouped(pf, x_sorted, w1, w3, w2, wgt):
    grid_spec = pltpu.PrefetchScalarGridSpec(
        num_scalar_prefetch=1,
        grid=(NT,),
        in_specs=[
            pl.BlockSpec((TILE, H), lambda i, pf: (i, 0)),
            pl.BlockSpec((1, FFN, H), lambda i, pf: (pf[0, i], 0, 0)),
            pl.BlockSpec((1, FFN, H), lambda i, pf: (pf[0, i], 0, 0)),
            pl.BlockSpec((1, H, FFN), lambda i, pf: (pf[0, i], 0, 0)),
            pl.BlockSpec((TILE, 1), lambda i, pf: (i, 0)),
        ],
        out_specs=pl.BlockSpec((TILE, H), lambda i, pf: (i, 0)),
    )
    return pl.pallas_call(
        _ffn_body,
        grid_spec=grid_spec,
        out_shape=jax.ShapeDtypeStruct((NP, H), jnp.float32),
        compiler_params=pltpu.CompilerParams(
            dimension_semantics=("arbitrary",),
        ),
    )(pf, x_sorted, w1, w3, w2, wgt)


_TOK_PER = T // NWORK
_CCH = 32


@functools.lru_cache(maxsize=None)
def _sc_combine_kernel():
    return functools.partial(
        pl.kernel,
        mesh=_sc_mesh(),
        out_type=jax.ShapeDtypeStruct((T, H), jnp.float32),
        scratch_types=[
            pltpu.VMEM((_TOK_PER,), jnp.int32),
            pltpu.VMEM((_TOK_PER,), jnp.int32),
            pltpu.VMEM((_CCH, H), jnp.float32),
            pltpu.VMEM((_CCH, H), jnp.float32),
            pltpu.SemaphoreType.DMA,
        ],
        compiler_params=pltpu.CompilerParams(needs_layout_passes=False),
    )(_sc_combine_body)


def _sc_combine(ys, p1, p2):
    return _sc_combine_kernel()(ys, p1, p2)


def _sc_combine_body(ys_hbm, p1_hbm, p2_hbm, out_hbm,
                     p1_v, p2_v, r1_v, r2_v, sem):
    wid = lax.axis_index("s") * 2 + lax.axis_index("c")
    base = wid * _TOK_PER
    pltpu.sync_copy(p1_hbm.at[pl.ds(base, _TOK_PER)], p1_v)
    pltpu.sync_copy(p2_hbm.at[pl.ds(base, _TOK_PER)], p2_v)

    def step(c, carry):
        pltpu.async_copy(
            ys_hbm.at[p1_v.at[pl.ds(c * _CCH, _CCH)]], r1_v, sem
        ).wait()
        pltpu.async_copy(
            ys_hbm.at[p2_v.at[pl.ds(c * _CCH, _CCH)]], r2_v, sem
        ).wait()

        def addrow(r, carry2):
            def addcol(k, carry3):
                sl = pl.ds(k * 16, 16)
                r1_v[r, sl] = r1_v[r, sl] + r2_v[r, sl]
                return carry3
            return lax.fori_loop(0, H // 16, addcol, carry2, unroll=8)

        lax.fori_loop(0, _CCH, addrow, 0)
        pltpu.sync_copy(r1_v, out_hbm.at[pl.ds(base + c * _CCH, _CCH)])
        return carry

    lax.fori_loop(0, _TOK_PER // _CCH, step, 0)


@jax.jit
def kernel(hidden_states, gate_w, w1, w2, w3):
    B, S, Hh = hidden_states.shape
    x = hidden_states.reshape(-1, Hh)
    gate_pad = jnp.zeros((LANES, Hh), jnp.float32).at[:E].set(gate_w)
    (logits_pad, pos1, pos2, wn1, wn2, te, valid) = _router(x, gate_pad)
    router_logits = logits_pad[:, :E]
    pos1f = pos1.reshape(T)
    pos2f = pos2.reshape(T)
    tok_sorted, wgt_sorted = _sc_meta(pos1f, pos2f,
                                      wn1.reshape(T), wn2.reshape(T))
    x_sorted = _sc_gather(x, tok_sorted)
    pf = jnp.concatenate([te[:NT, 0][None, :], valid[:NT, 0][None, :]], axis=0)
    y_sorted = _ffn_grouped(pf, x_sorted, w1, w3, w2,
                            wgt_sorted.reshape(NP, 1))
    final = _sc_combine(y_sorted, pos1f, pos2f)
    return final.reshape(B, S, Hh), router_logits
